# Initial kernel scaffold; baseline (speedup 1.0000x reference)
import jax
import jax.numpy as jnp
from jax.experimental import pallas as pl


def kernel(X, mu, niter):
    raise NotImplementedError("write your pallas kernel here")


# speedup vs baseline: 2.6866x; 2.6866x over previous
import functools

import jax
import jax.numpy as jnp
from jax.experimental import pallas as pl
from jax.experimental.pallas import tpu as pltpu


_BN = 1024


def _assign_reduce_body(x_ref, m_ref, sums_ref, cnt_ref):
    i = pl.program_id(0)
    x = x_ref[...]
    m = m_ref[...]
    bn = x.shape[0]
    nc = m.shape[0]

    x2 = jnp.sum(x * x, axis=1)
    m2 = jnp.sum(m * m, axis=1)
    dot = jax.lax.dot_general(
        x, m, (((1,), (1,)), ((), ())),
        preferred_element_type=jnp.float32)
    d2 = (x2[:, None] + m2[None, :]) - 2.0 * dot
    dist = jnp.sqrt(jnp.maximum(d2, 0.0))
    mn = jnp.min(dist, axis=1, keepdims=True)
    lane = jax.lax.broadcasted_iota(jnp.int32, (bn, nc), 1)
    idx = jnp.min(jnp.where(dist == mn, lane, nc), axis=1)

    oh = (idx[:, None] == lane).astype(jnp.float32)
    psum = jax.lax.dot_general(
        oh, x, (((0,), (0,)), ((), ())),
        preferred_element_type=jnp.float32,
        precision=jax.lax.Precision.HIGHEST)
    pcnt = jnp.sum(oh, axis=0)

    @pl.when(i == 0)
    def _():
        sums_ref[...] = psum
        cnt_ref[...] = pcnt

    @pl.when(i > 0)
    def _():
        sums_ref[...] += psum
        cnt_ref[...] += pcnt


def _update_body(m_ref, sums_ref, cnt_ref, out_ref):
    c = cnt_ref[...]
    s = sums_ref[...]
    m = m_ref[...]
    mu_new = s / jnp.maximum(c, 1.0)[:, None]
    out_ref[...] = jnp.where(c[:, None] > 0, mu_new, m)


@functools.partial(jax.jit, static_argnames=("interpret",))
def _one_iter(Xr, M, interpret=False):
    n, k = Xr.shape
    nc = M.shape[0]
    nb = n // _BN
    sums, counts = pl.pallas_call(
        _assign_reduce_body,
        grid=(nb,),
        in_specs=[
            pl.BlockSpec((_BN, k), lambda i: (i, 0)),
            pl.BlockSpec((nc, k), lambda i: (0, 0)),
        ],
        out_specs=[
            pl.BlockSpec((nc, k), lambda i: (0, 0)),
            pl.BlockSpec((nc,), lambda i: (0,)),
        ],
        out_shape=[
            jax.ShapeDtypeStruct((nc, k), jnp.float32),
            jax.ShapeDtypeStruct((nc,), jnp.float32),
        ],
        interpret=interpret,
    )(Xr, M)
    return pl.pallas_call(
        _update_body,
        interpret=interpret,
        out_shape=jax.ShapeDtypeStruct((nc, k), jnp.float32),
    )(M, sums, counts)


def kernel(X, mu, niter):
    nc, _, k = mu.shape
    Xr = X.reshape(-1, k)
    M0 = mu[:, 0, :]
    Mf = jax.lax.fori_loop(0, niter, lambda t, M: _one_iter(Xr, M), M0)
    return Mf[:, None, :]
